# baseline (speedup 1.0000x reference)
import functools

import jax
import jax.numpy as jnp
from jax import lax
from jax.experimental import pallas as pl
from jax.experimental.pallas import tpu as pltpu
from jax.experimental.pallas import tpu_sc as plsc

_B, _L = 4096, 200
_NC, _NS = 2, 16
_NW = _NC * _NS
_COLS_W = _B // _NW

_mesh = plsc.VectorSubcoreMesh(core_axis_name="c", subcore_axis_name="s")


@functools.partial(
    pl.kernel,
    mesh=_mesh,
    out_type=jax.ShapeDtypeStruct((_L, _B), jnp.float32),
    compiler_params=pltpu.CompilerParams(use_tc_tiling_on_sc=True),
    scratch_types=[
        pltpu.VMEM((_L, _COLS_W), jnp.int32),
        pltpu.VMEM((_L, _COLS_W), jnp.float32),
        pltpu.SemaphoreType.DMA,
        pltpu.SemaphoreType.DMA,
    ],
)
def _lookup(idx_hbm, table_hbm, out_hbm, idx_v, rows_v, sem, sem2):
    wid = lax.axis_index("s") * _NC + lax.axis_index("c")
    c0 = wid * _COLS_W
    _G = 50
    idx_r = idx_v.reshape(4, _G * _COLS_W)
    rows_r = rows_v.reshape(4, _G * _COLS_W)

    _STAGE = ((0, 56), (56, 48), (104, 48), (152, 48))
    for g in range(4):
        t0, tn = _STAGE[g]
        pltpu.sync_copy(idx_hbm.at[pl.ds(t0, tn), pl.ds(c0, _COLS_W)],
                        idx_v.at[pl.ds(t0, tn), :])
        pltpu.async_copy(table_hbm.at[idx_r.at[g]], rows_r.at[g], sem)

    _S = ((0, 48), (48, 48), (96, 48), (144, 56))
    for g in range(4):
        pltpu.make_async_copy(
            table_hbm.at[idx_r.at[g]], rows_r.at[g], sem
        ).wait()
        s0, sn = _S[g]
        pltpu.async_copy(
            rows_v.at[pl.ds(s0, sn), :],
            out_hbm.at[pl.ds(s0, sn), pl.ds(c0, _COLS_W)],
            sem2,
        )
    for g in range(4):
        s0, sn = _S[g]
        pltpu.make_async_copy(
            rows_v.at[pl.ds(s0, sn), :],
            out_hbm.at[pl.ds(s0, sn), pl.ds(c0, _COLS_W)],
            sem2,
        ).wait()


def kernel(indices, table):
    out_t = _lookup(indices.T, table)
    return out_t.T


# speedup vs baseline: 1.0044x; 1.0044x over previous
import functools

import jax
import jax.numpy as jnp
from jax import lax
from jax.experimental import pallas as pl
from jax.experimental.pallas import tpu as pltpu
from jax.experimental.pallas import tpu_sc as plsc

_B, _L = 4096, 200
_NC, _NS = 2, 16
_NW = _NC * _NS
_COLS_W = _B // _NW

_mesh = plsc.VectorSubcoreMesh(core_axis_name="c", subcore_axis_name="s")


@functools.partial(
    pl.kernel,
    mesh=_mesh,
    out_type=jax.ShapeDtypeStruct((_L, _B), jnp.float32),
    compiler_params=pltpu.CompilerParams(use_tc_tiling_on_sc=True),
    scratch_types=[
        pltpu.VMEM((_L, _COLS_W), jnp.int32),
        pltpu.VMEM((_L, _COLS_W), jnp.float32),
        pltpu.SemaphoreType.DMA,
        pltpu.SemaphoreType.DMA,
    ],
)
def _lookup(idx_hbm, table_hbm, out_hbm, idx_v, rows_v, sem, sem2):
    wid = lax.axis_index("s") * _NC + lax.axis_index("c")
    c0 = wid * _COLS_W
    _H1 = 104
    _H2 = _L - _H1
    _G = 100
    idx_r = idx_v.reshape(2, _G * _COLS_W)
    rows_r = rows_v.reshape(2, _G * _COLS_W)

    pltpu.sync_copy(idx_hbm.at[pl.ds(0, _H1), pl.ds(c0, _COLS_W)],
                    idx_v.at[pl.ds(0, _H1), :])
    pltpu.async_copy(table_hbm.at[idx_r.at[0]], rows_r.at[0], sem)
    pltpu.sync_copy(idx_hbm.at[pl.ds(_H1, _H2), pl.ds(c0, _COLS_W)],
                    idx_v.at[pl.ds(_H1, _H2), :])
    pltpu.async_copy(table_hbm.at[idx_r.at[1]], rows_r.at[1], sem)

    _S = ((0, 96), (96, 104))
    for g in range(2):
        pltpu.make_async_copy(
            table_hbm.at[idx_r.at[g]], rows_r.at[g], sem
        ).wait()
        s0, sn = _S[g]
        pltpu.async_copy(
            rows_v.at[pl.ds(s0, sn), :],
            out_hbm.at[pl.ds(s0, sn), pl.ds(c0, _COLS_W)],
            sem2,
        )
    for g in range(2):
        s0, sn = _S[g]
        pltpu.make_async_copy(
            rows_v.at[pl.ds(s0, sn), :],
            out_hbm.at[pl.ds(s0, sn), pl.ds(c0, _COLS_W)],
            sem2,
        ).wait()


def kernel(indices, table):
    out_t = _lookup(indices.T, table)
    return out_t.T
